# Initial kernel scaffold; baseline (speedup 1.0000x reference)
import jax
import jax.numpy as jnp
from jax.experimental import pallas as pl


def kernel(out, tokens, W):
    raise NotImplementedError("write your pallas kernel here")


# speedup vs baseline: 2.8879x; 2.8879x over previous
import jax
import jax.numpy as jnp
from jax.experimental import pallas as pl

B = 8
TH = 8192
D = 256
STRIDE = 4
U = 6
H = TH * STRIDE

TBLK = 1024
NT = TH // TBLK


def _fold_kernel(tok_ref, halo_ref, w_ref, out_ref):
    i = pl.program_id(1)
    tok = tok_ref[0]
    w = w_ref[...]
    proj = jnp.dot(tok, w, preferred_element_type=jnp.float32)
    y = proj[:, U:2 * U] * jax.nn.sigmoid(proj[:, 0:U])

    hrow = halo_ref[0, 7:8, :]
    hproj = jnp.dot(hrow, w, preferred_element_type=jnp.float32)
    hy45 = hproj[:, U + 4:U + 6] * jax.nn.sigmoid(hproj[:, 4:6])

    prev45 = jnp.concatenate([hy45, y[:-1, 4:6]], axis=0)

    grow = jax.lax.broadcasted_iota(jnp.int32, (TBLK, 4), 0) + i * TBLK
    col = jax.lax.broadcasted_iota(jnp.int32, (TBLK, 4), 1)
    grow2 = grow[:, 0:1]

    head = y[:, 0:2] + jnp.where(grow2 > 0, prev45, 0.0)
    sums = jnp.concatenate([head, y[:, 2:4]], axis=1)

    tailv = y[TBLK - 1:TBLK, 4:5] + y[TBLK - 1:TBLK, 5:6]
    sums = sums + jnp.where((grow == TH - 1) & (col == 3), tailv, 0.0)

    inv = jnp.where((col < 2) & (grow > 0), 0.5, 1.0)
    inv = jnp.where((grow == TH - 1) & (col == 3), 1.0 / 3.0, inv)
    out_ref[0] = sums * inv


def kernel(out, tokens, W):
    del out
    wmat = jnp.concatenate([W[0].T, W[1].T], axis=1)
    res4 = pl.pallas_call(
        _fold_kernel,
        grid=(B, NT),
        in_specs=[
            pl.BlockSpec((1, TBLK, D), lambda b, i: (b, i, 0)),
            pl.BlockSpec((1, 8, D),
                         lambda b, i: (b, jnp.maximum(i * (TBLK // 8) - 1, 0), 0)),
            pl.BlockSpec((D, 2 * U), lambda b, i: (0, 0)),
        ],
        out_specs=pl.BlockSpec((1, TBLK, 4), lambda b, i: (b, i, 0)),
        out_shape=jax.ShapeDtypeStruct((B, TH, 4), jnp.float32),
    )(tokens, tokens, wmat)
    return res4.reshape(B, H)
